# baseline (speedup 1.0000x reference)
import jax
import jax.numpy as jnp
from jax.experimental import pallas as pl
from jax.experimental.pallas import tpu as pltpu
from jax.experimental.pallas import tpu_sc as plsc

_NUM_SC_WORKERS = 32


def _tile_kernel(x_ref, col_ref, row_ref, tile_ref):
    num_clips, d = col_ref.shape
    hw = x_ref.shape[1]
    w = hw // num_clips

    k_iota = jax.lax.broadcasted_iota(jnp.int32, (num_clips, hw), 0)
    p_iota = jax.lax.broadcasted_iota(jnp.int32, (num_clips, hw), 1)
    onehot_col = (x_ref[:] == k_iota).astype(jnp.float32)
    onehot_row = ((p_iota // w) == k_iota).astype(jnp.float32)

    dn = (((0,), (0,)), ((), ()))
    tile_ref[:d, :] = jax.lax.dot_general(col_ref[:], onehot_col, dn,
                                          preferred_element_type=jnp.float32,
                                          precision=jax.lax.Precision.HIGHEST)
    tile_ref[d:, :] = jax.lax.dot_general(row_ref[:], onehot_row, dn,
                                          preferred_element_type=jnp.float32,
                                          precision=jax.lax.Precision.HIGHEST)


def kernel(x, col_embed, row_embed):
    h, w = x.shape
    num_clips, d = col_embed.shape
    b = h
    hw = h * w
    rows = 2 * d // _NUM_SC_WORKERS

    x_flat = x.reshape(1, hw)

    tile = pl.pallas_call(
        _tile_kernel,
        out_shape=jax.ShapeDtypeStruct((2 * d, hw), jnp.float32),
    )(x_flat, col_embed, row_embed)

    sc_mesh = plsc.VectorSubcoreMesh(core_axis_name="c", subcore_axis_name="s")

    @pl.kernel(
        out_type=jax.ShapeDtypeStruct((b, 2 * d, hw), jnp.float32),
        mesh=sc_mesh,
        scratch_types=[
            pltpu.VMEM((rows, hw), jnp.float32),
            pltpu.SemaphoreType.DMA,
            pltpu.SemaphoreType.DMA,
        ],
        compiler_params=pltpu.CompilerParams(use_tc_tiling_on_sc=True),
    )
    def _broadcast_kernel(tile_ref, out_ref, chunk, sem_in, sem_out):
        core = jax.lax.axis_index("c")
        sub = jax.lax.axis_index("s")
        tid = core * 16 + sub
        r0 = tid * rows
        pltpu.async_copy(tile_ref.at[pl.ds(r0, rows), :], chunk, sem_in).wait()
        copies = [
            pltpu.make_async_copy(chunk, out_ref.at[i, pl.ds(r0, rows), :],
                                  sem_out)
            for i in range(b)
        ]
        for c in copies:
            c.start()
        for c in copies:
            c.wait()

    out_flat = _broadcast_kernel(tile)
    return out_flat.reshape(b, 2 * d, h, w)


# speedup vs baseline: 3.8569x; 3.8569x over previous
import jax
import jax.numpy as jnp
from jax.experimental import pallas as pl
from jax.experimental.pallas import tpu as pltpu


def _pos_embed_kernel(x_ref, col_ref, row_ref, out_ref, tile, sems):
    num_clips, d = col_ref.shape
    hw = x_ref.shape[0]
    w = hw // num_clips
    b = out_ref.shape[0]

    k_iota = jax.lax.broadcasted_iota(jnp.int32, (hw, num_clips), 1)
    p_iota = jax.lax.broadcasted_iota(jnp.int32, (hw, num_clips), 0)

    onehot_col = (x_ref[:] == k_iota).astype(jnp.float32)
    onehot_row = ((p_iota // w) == k_iota).astype(jnp.float32)

    dn = (((1,), (0,)), ((), ()))
    tile[:, :d] = jax.lax.dot_general(onehot_col, col_ref[:], dn,
                                      preferred_element_type=jnp.float32,
                                      precision=jax.lax.Precision.HIGHEST)
    tile[:, d:] = jax.lax.dot_general(onehot_row, row_ref[:], dn,
                                      preferred_element_type=jnp.float32,
                                      precision=jax.lax.Precision.HIGHEST)

    copies = [pltpu.make_async_copy(tile, out_ref.at[i], sems.at[i])
              for i in range(b)]
    for c in copies:
        c.start()
    for c in copies:
        c.wait()


def kernel(x, col_embed, row_embed):
    h, w = x.shape
    num_clips, d = col_embed.shape
    b = h
    hw = h * w

    x_col = x.reshape(hw, 1)

    out_pm = pl.pallas_call(
        _pos_embed_kernel,
        in_specs=[
            pl.BlockSpec(memory_space=pltpu.MemorySpace.VMEM),
            pl.BlockSpec(memory_space=pltpu.MemorySpace.VMEM),
            pl.BlockSpec(memory_space=pltpu.MemorySpace.VMEM),
        ],
        out_specs=pl.BlockSpec(memory_space=pltpu.MemorySpace.HBM),
        out_shape=jax.ShapeDtypeStruct((b, hw, 2 * d), jnp.float32),
        scratch_shapes=[
            pltpu.VMEM((hw, 2 * d), jnp.float32),
            pltpu.SemaphoreType.DMA((b,)),
        ],
    )(x_col, col_embed, row_embed)

    return out_pm.reshape(b, h, w, 2 * d).transpose(0, 3, 1, 2)


# speedup vs baseline: 3.9770x; 1.0311x over previous
import jax
import jax.numpy as jnp
from jax.experimental import pallas as pl
from jax.experimental.pallas import tpu as pltpu


def _pos_embed_kernel(x_ref, col_ref, row_ref, out_ref, tile, sems):
    num_clips, d = col_ref.shape
    hw = x_ref.shape[0]
    w = hw // num_clips
    b = out_ref.shape[0]

    k_iota = jax.lax.broadcasted_iota(jnp.int32, (hw, num_clips), 1)
    p_iota = jax.lax.broadcasted_iota(jnp.int32, (hw, num_clips), 0)

    onehot_col = (x_ref[:] == k_iota).astype(jnp.float32)
    onehot_row = ((p_iota // w) == k_iota).astype(jnp.float32)

    dn = (((1,), (0,)), ((), ()))

    n_blocks = 8
    rows = hw // n_blocks
    copies = []
    for blk in range(n_blocks):
        rs = pl.ds(blk * rows, rows)
        tile[rs, :d] = jax.lax.dot_general(
            onehot_col[blk * rows:(blk + 1) * rows, :], col_ref[:], dn,
            preferred_element_type=jnp.float32,
            precision=jax.lax.Precision.HIGHEST)
        tile[rs, d:] = jax.lax.dot_general(
            onehot_row[blk * rows:(blk + 1) * rows, :], row_ref[:], dn,
            preferred_element_type=jnp.float32,
            precision=jax.lax.Precision.HIGHEST)
        for i in range(b):
            c = pltpu.make_async_copy(tile.at[rs], out_ref.at[i, rs],
                                      sems.at[i])
            c.start()
            copies.append(c)
    for c in copies:
        c.wait()


def kernel(x, col_embed, row_embed):
    h, w = x.shape
    num_clips, d = col_embed.shape
    b = h
    hw = h * w

    x_col = x.reshape(hw, 1)

    out_pm = pl.pallas_call(
        _pos_embed_kernel,
        in_specs=[
            pl.BlockSpec(memory_space=pltpu.MemorySpace.VMEM),
            pl.BlockSpec(memory_space=pltpu.MemorySpace.VMEM),
            pl.BlockSpec(memory_space=pltpu.MemorySpace.VMEM),
        ],
        out_specs=pl.BlockSpec(memory_space=pltpu.MemorySpace.HBM),
        out_shape=jax.ShapeDtypeStruct((b, hw, 2 * d), jnp.float32),
        scratch_shapes=[
            pltpu.VMEM((hw, 2 * d), jnp.float32),
            pltpu.SemaphoreType.DMA((b,)),
        ],
    )(x_col, col_embed, row_embed)

    return out_pm.reshape(b, h, w, 2 * d).transpose(0, 3, 1, 2)
